# Initial kernel scaffold; baseline (speedup 1.0000x reference)
import jax
import jax.numpy as jnp
from jax.experimental import pallas as pl


def kernel(x, W_q, b_q, bn_gamma, bn_beta, c1, c2, values):
    raise NotImplementedError("write your pallas kernel here")


# speedup vs baseline: 1.8439x; 1.8439x over previous
import functools

import jax
import jax.numpy as jnp
from jax import lax
from jax.experimental import pallas as pl
from jax.experimental.pallas import tpu as pltpu
from jax.experimental.pallas import tpu_sc as plsc

D_MODEL = 1024
QDIM = 512
NSUB = 256
K = 16
H = 4
SEQ = 2048
TOK_BLK = 256
N_BLK = SEQ // TOK_BLK
KH = K * H


def _proj_body(x_ref, w_ref, b_ref, q_ref, s1_ref, s2_ref):
    q = jnp.dot(x_ref[...], w_ref[...], preferred_element_type=jnp.float32)
    q = q + b_ref[...]
    q_ref[...] = q
    s1_ref[...] = jnp.sum(q, axis=0, keepdims=True)[None]
    s2_ref[...] = jnp.sum(q * q, axis=0, keepdims=True)[None]


def _tc_proj(x2d, W_q, b_q2d):
    return pl.pallas_call(
        _proj_body,
        grid=(N_BLK,),
        in_specs=[
            pl.BlockSpec((TOK_BLK, D_MODEL), lambda i: (i, 0)),
            pl.BlockSpec((D_MODEL, H * QDIM), lambda i: (0, 0)),
            pl.BlockSpec((1, H * QDIM), lambda i: (0, 0)),
        ],
        out_specs=[
            pl.BlockSpec((TOK_BLK, H * QDIM), lambda i: (i, 0)),
            pl.BlockSpec((1, 1, H * QDIM), lambda i: (i, 0, 0)),
            pl.BlockSpec((1, 1, H * QDIM), lambda i: (i, 0, 0)),
        ],
        out_shape=[
            jax.ShapeDtypeStruct((SEQ, H * QDIM), jnp.float32),
            jax.ShapeDtypeStruct((N_BLK, 1, H * QDIM), jnp.float32),
            jax.ShapeDtypeStruct((N_BLK, 1, H * QDIM), jnp.float32),
        ],
    )(x2d, W_q, b_q2d)


def _topk16(s, iota):
    T = s.shape[0]
    lane16 = lax.broadcasted_iota(jnp.int32, (T, K), 1)
    ts = jnp.zeros((T, K), jnp.float32)
    ti = jnp.zeros((T, K), jnp.int32)
    for k in range(K):
        m = jnp.max(s, axis=1, keepdims=True)
        hit = s == m
        arg = jnp.min(jnp.where(hit, iota, NSUB), axis=1, keepdims=True)
        ts = jnp.where(lane16 == k, m, ts)
        ti = jnp.where(lane16 == k, arg, ti)
        s = jnp.where(iota == arg, -jnp.inf, s)
    return ts, ti


def _scores_body(q_ref, s1_ref, s2_ref, g_ref, bb_ref, c1_ref, c2_ref,
                 midx_ref, attn_ref):
    inv_n = jnp.float32(1.0 / SEQ)
    mean = jnp.sum(s1_ref[...][:, 0, :], axis=0, keepdims=True) * inv_n
    ex2 = jnp.sum(s2_ref[...][:, 0, :], axis=0, keepdims=True) * inv_n
    var = ex2 - mean * mean
    scale = g_ref[...] / jnp.sqrt(var + 1e-5)
    shift = bb_ref[...] - mean * scale
    qn = q_ref[...] * scale + shift

    T = TOK_BLK
    iota = lax.broadcasted_iota(jnp.int32, (T, NSUB), 1)
    ri = lax.broadcasted_iota(jnp.int32, (K, NSUB), 0)
    ci = lax.broadcasted_iota(jnp.int32, (K, NSUB), 1)
    R1 = (ci // K == ri).astype(jnp.float32)
    R2 = (ci % K == ri).astype(jnp.float32)

    q1 = qn[:, :NSUB]
    q2 = qn[:, NSUB:]
    sc1 = lax.dot_general(q1, c1_ref[0], (((1,), (1,)), ((), ())),
                          preferred_element_type=jnp.float32)
    sc2 = lax.dot_general(q2, c2_ref[0], (((1,), (1,)), ((), ())),
                          preferred_element_type=jnp.float32)
    ts1, ti1 = _topk16(sc1, iota)
    ts2, ti2 = _topk16(sc2, iota)
    all_s = (jnp.dot(ts1, R1, preferred_element_type=jnp.float32,
                     precision=lax.Precision.HIGHEST)
             + jnp.dot(ts2, R2, preferred_element_type=jnp.float32,
                       precision=lax.Precision.HIGHEST))
    gs, gi = _topk16(all_s, iota)
    p1 = gi // K
    p2 = gi % K
    real1 = jnp.zeros((T, K), jnp.int32)
    real2 = jnp.zeros((T, K), jnp.int32)
    for p in range(K):
        real1 = real1 + jnp.where(p1 == p, ti1[:, p:p + 1], 0)
        real2 = real2 + jnp.where(p2 == p, ti2[:, p:p + 1], 0)
    mem = real1 * NSUB + real2
    m = jnp.max(gs, axis=1, keepdims=True)
    e = jnp.exp(gs - m)
    w = e / jnp.sum(e, axis=1, keepdims=True)
    midx_ref[0] = mem
    attn_ref[0] = w


def _tc_scores(q, s1, s2, gamma2d, beta2d, c1, c2):
    return pl.pallas_call(
        _scores_body,
        grid=(H, N_BLK),
        in_specs=[
            pl.BlockSpec((TOK_BLK, QDIM), lambda h, i: (i, h)),
            pl.BlockSpec((N_BLK, 1, QDIM), lambda h, i: (0, 0, h)),
            pl.BlockSpec((N_BLK, 1, QDIM), lambda h, i: (0, 0, h)),
            pl.BlockSpec((1, QDIM), lambda h, i: (0, h)),
            pl.BlockSpec((1, QDIM), lambda h, i: (0, h)),
            pl.BlockSpec((1, NSUB, QDIM // 2), lambda h, i: (h, 0, 0)),
            pl.BlockSpec((1, NSUB, QDIM // 2), lambda h, i: (h, 0, 0)),
        ],
        out_specs=[
            pl.BlockSpec((1, TOK_BLK, K), lambda h, i: (h, i, 0)),
            pl.BlockSpec((1, TOK_BLK, K), lambda h, i: (h, i, 0)),
        ],
        out_shape=[
            jax.ShapeDtypeStruct((H, SEQ, K), jnp.int32),
            jax.ShapeDtypeStruct((H, SEQ, K), jnp.float32),
        ],
    )(q, s1, s2, gamma2d, beta2d, c1, c2)


TOK_PER_TILE = SEQ // 32


HALF = KH // 2


def _sc_body(values_hbm, idx_hbm, w_hbm, out_hbm, idx_v, w_v, buf0, buf1,
             acc_v, sem0, sem1):
    wid = lax.axis_index("s") * 2 + lax.axis_index("c")
    base = wid * TOK_PER_TILE
    pltpu.sync_copy(idx_hbm.at[pl.ds(base, TOK_PER_TILE)], idx_v)
    pltpu.sync_copy(w_hbm.at[pl.ds(base, TOK_PER_TILE)], w_v)

    def start(t, half, buf, sem):
        pltpu.async_copy(
            values_hbm.at[idx_v.at[t, pl.ds(half * HALF, HALF)]], buf, sem)

    def wait(buf, sem):
        pltpu.make_async_copy(values_hbm.at[pl.ds(0, HALF)], buf, sem).wait()

    def combine(buf, t, half, first):
        for kc in range(HALF // K):
            w_vec = w_v[t, pl.ds(half * HALF + kc * K, K)]
            ws = [w_vec[i] for i in range(K)]

            def j_body(j, c, kc=kc, ws=ws, overwrite=(first and kc == 0)):
                sl = pl.ds(j * 16, 16)
                parts = [ws[q] * buf[kc * K + q, sl] for q in range(4)]
                for i in range(4, K):
                    q = i % 4
                    parts[q] = parts[q] + ws[i] * buf[kc * K + i, sl]
                part = (parts[0] + parts[1]) + (parts[2] + parts[3])
                if overwrite:
                    acc_v[0, sl] = part
                else:
                    acc_v[0, sl] += part
                return c

            lax.fori_loop(0, D_MODEL // 16, j_body, 0)

    start(0, 0, buf0, sem0)

    def tok_body(t, carry):
        wait(buf0, sem0)
        start(t, 1, buf1, sem1)
        combine(buf0, t, 0, True)
        wait(buf1, sem1)
        start(jnp.minimum(t + 1, TOK_PER_TILE - 1), 0, buf0, sem0)
        combine(buf1, t, 1, False)
        pltpu.sync_copy(acc_v, out_hbm.at[pl.ds(base + t, 1)])
        return carry

    lax.fori_loop(0, TOK_PER_TILE, tok_body, 0)
    wait(buf0, sem0)


def _sc_combine(values, midx, attn):
    mesh = plsc.VectorSubcoreMesh(core_axis_name="c", subcore_axis_name="s")
    f = functools.partial(
        pl.kernel,
        mesh=mesh,
        out_type=jax.ShapeDtypeStruct((SEQ, D_MODEL), jnp.float32),
        scratch_types=[
            pltpu.VMEM((TOK_PER_TILE, KH), jnp.int32),
            pltpu.VMEM((TOK_PER_TILE, KH), jnp.float32),
            pltpu.VMEM((HALF, D_MODEL), jnp.float32),
            pltpu.VMEM((HALF, D_MODEL), jnp.float32),
            pltpu.VMEM((1, D_MODEL), jnp.float32),
            pltpu.SemaphoreType.DMA,
            pltpu.SemaphoreType.DMA,
        ],
    )(_sc_body)
    return f(values, midx, attn)


def kernel(x, W_q, b_q, bn_gamma, bn_beta, c1, c2, values):
    bs, seq_len, d_model = x.shape
    x2d = x.reshape(seq_len, d_model)
    q, s1, s2 = _tc_proj(x2d, W_q, b_q.reshape(1, -1))
    midx_h, attn_h = _tc_scores(q, s1, s2, bn_gamma.reshape(1, -1),
                                bn_beta.reshape(1, -1), c1, c2)
    midx = jnp.transpose(midx_h, (1, 0, 2)).reshape(SEQ, KH)
    attn = jnp.transpose(attn_h, (1, 0, 2)).reshape(SEQ, KH)
    out = _sc_combine(values, midx, attn)
    return out.reshape(bs, seq_len, d_model)


# speedup vs baseline: 1.9272x; 1.0452x over previous
import functools

import jax
import jax.numpy as jnp
from jax import lax
from jax.experimental import pallas as pl
from jax.experimental.pallas import tpu as pltpu
from jax.experimental.pallas import tpu_sc as plsc

D_MODEL = 1024
QDIM = 512
NSUB = 256
K = 16
H = 4
SEQ = 2048
TOK_BLK = 256
N_BLK = SEQ // TOK_BLK
KH = K * H


def _proj_body(x_ref, w_ref, b_ref, q_ref, s1_ref, s2_ref):
    q = jnp.dot(x_ref[...], w_ref[...], preferred_element_type=jnp.float32)
    q = q + b_ref[...]
    q_ref[...] = q
    s1_ref[...] = jnp.sum(q, axis=0, keepdims=True)[None]
    s2_ref[...] = jnp.sum(q * q, axis=0, keepdims=True)[None]


def _tc_proj(x2d, W_q, b_q2d):
    return pl.pallas_call(
        _proj_body,
        grid=(N_BLK,),
        in_specs=[
            pl.BlockSpec((TOK_BLK, D_MODEL), lambda i: (i, 0)),
            pl.BlockSpec((D_MODEL, H * QDIM), lambda i: (0, 0)),
            pl.BlockSpec((1, H * QDIM), lambda i: (0, 0)),
        ],
        out_specs=[
            pl.BlockSpec((TOK_BLK, H * QDIM), lambda i: (i, 0)),
            pl.BlockSpec((1, 1, H * QDIM), lambda i: (i, 0, 0)),
            pl.BlockSpec((1, 1, H * QDIM), lambda i: (i, 0, 0)),
        ],
        out_shape=[
            jax.ShapeDtypeStruct((SEQ, H * QDIM), jnp.float32),
            jax.ShapeDtypeStruct((N_BLK, 1, H * QDIM), jnp.float32),
            jax.ShapeDtypeStruct((N_BLK, 1, H * QDIM), jnp.float32),
        ],
    )(x2d, W_q, b_q2d)


def _topk16(s, iota):
    T = s.shape[0]
    lane16 = lax.broadcasted_iota(jnp.int32, (T, K), 1)
    ts = jnp.zeros((T, K), jnp.float32)
    ti = jnp.zeros((T, K), jnp.int32)
    for k in range(K):
        m = jnp.max(s, axis=1, keepdims=True)
        hit = s == m
        arg = jnp.min(jnp.where(hit, iota, NSUB), axis=1, keepdims=True)
        ts = jnp.where(lane16 == k, m, ts)
        ti = jnp.where(lane16 == k, arg, ti)
        s = jnp.where(iota == arg, -jnp.inf, s)
    return ts, ti


def _scores_body(q_ref, s1_ref, s2_ref, g_ref, bb_ref, c1_ref, c2_ref,
                 midx_ref, attn_ref):
    inv_n = jnp.float32(1.0 / SEQ)
    mean = jnp.sum(s1_ref[...][:, 0, :], axis=0, keepdims=True) * inv_n
    ex2 = jnp.sum(s2_ref[...][:, 0, :], axis=0, keepdims=True) * inv_n
    var = ex2 - mean * mean
    scale = g_ref[...] / jnp.sqrt(var + 1e-5)
    shift = bb_ref[...] - mean * scale
    qn = q_ref[...] * scale + shift

    T = TOK_BLK
    iota = lax.broadcasted_iota(jnp.int32, (T, NSUB), 1)
    ri = lax.broadcasted_iota(jnp.int32, (K, NSUB), 0)
    ci = lax.broadcasted_iota(jnp.int32, (K, NSUB), 1)
    R1 = (ci // K == ri).astype(jnp.float32)
    R2 = (ci % K == ri).astype(jnp.float32)

    q1 = qn[:, :NSUB]
    q2 = qn[:, NSUB:]
    sc1 = lax.dot_general(q1, c1_ref[0], (((1,), (1,)), ((), ())),
                          preferred_element_type=jnp.float32)
    sc2 = lax.dot_general(q2, c2_ref[0], (((1,), (1,)), ((), ())),
                          preferred_element_type=jnp.float32)
    ts1, ti1 = _topk16(sc1, iota)
    ts2, ti2 = _topk16(sc2, iota)
    all_s = (jnp.dot(ts1, R1, preferred_element_type=jnp.float32,
                     precision=lax.Precision.HIGHEST)
             + jnp.dot(ts2, R2, preferred_element_type=jnp.float32,
                       precision=lax.Precision.HIGHEST))
    gs, gi = _topk16(all_s, iota)
    p1 = gi // K
    p2 = gi % K
    real1 = jnp.zeros((T, K), jnp.int32)
    real2 = jnp.zeros((T, K), jnp.int32)
    for p in range(K):
        real1 = real1 + jnp.where(p1 == p, ti1[:, p:p + 1], 0)
        real2 = real2 + jnp.where(p2 == p, ti2[:, p:p + 1], 0)
    mem = real1 * NSUB + real2
    m = jnp.max(gs, axis=1, keepdims=True)
    e = jnp.exp(gs - m)
    w = e / jnp.sum(e, axis=1, keepdims=True)
    midx_ref[0] = mem
    attn_ref[0] = w


def _tc_scores(q, s1, s2, gamma2d, beta2d, c1, c2):
    return pl.pallas_call(
        _scores_body,
        grid=(H, N_BLK),
        in_specs=[
            pl.BlockSpec((TOK_BLK, QDIM), lambda h, i: (i, h)),
            pl.BlockSpec((N_BLK, 1, QDIM), lambda h, i: (0, 0, h)),
            pl.BlockSpec((N_BLK, 1, QDIM), lambda h, i: (0, 0, h)),
            pl.BlockSpec((1, QDIM), lambda h, i: (0, h)),
            pl.BlockSpec((1, QDIM), lambda h, i: (0, h)),
            pl.BlockSpec((1, NSUB, QDIM // 2), lambda h, i: (h, 0, 0)),
            pl.BlockSpec((1, NSUB, QDIM // 2), lambda h, i: (h, 0, 0)),
        ],
        out_specs=[
            pl.BlockSpec((1, TOK_BLK, K), lambda h, i: (h, i, 0)),
            pl.BlockSpec((1, TOK_BLK, K), lambda h, i: (h, i, 0)),
        ],
        out_shape=[
            jax.ShapeDtypeStruct((H, SEQ, K), jnp.int32),
            jax.ShapeDtypeStruct((H, SEQ, K), jnp.float32),
        ],
    )(q, s1, s2, gamma2d, beta2d, c1, c2)


TOK_PER_TILE = SEQ // 32


HALF = KH // 2


def _sc_body(values_hbm, idx_hbm, w_hbm, out_hbm, idx_v, w_v, buf0, buf1,
             acc_v, sem0, sem1):
    wid = lax.axis_index("s") * 2 + lax.axis_index("c")
    base = wid * TOK_PER_TILE
    pltpu.sync_copy(idx_hbm.at[pl.ds(base, TOK_PER_TILE)], idx_v)
    pltpu.sync_copy(w_hbm.at[pl.ds(base, TOK_PER_TILE)], w_v)

    def start(t, half, buf, sem):
        pltpu.async_copy(
            values_hbm.at[idx_v.at[t, pl.ds(half * HALF, HALF)]], buf, sem)

    def wait(buf, sem):
        pltpu.make_async_copy(values_hbm.at[pl.ds(0, HALF)], buf, sem).wait()

    def combine(buf, t, half, first):
        w0 = w_v[t, pl.ds(half * HALF, K)]
        w1 = w_v[t, pl.ds(half * HALF + K, K)]
        ws = [w0[i] for i in range(K)] + [w1[i] for i in range(K)]

        @plsc.parallel_loop(0, D_MODEL // 16, unroll=2)
        def j_body(j):
            sl = pl.ds(j * 16, 16)
            parts = [ws[q] * buf[q, sl] for q in range(4)]
            for i in range(4, HALF):
                q = i % 4
                parts[q] = parts[q] + ws[i] * buf[i, sl]
            part = (parts[0] + parts[1]) + (parts[2] + parts[3])
            if first:
                acc_v[0, sl] = part
            else:
                acc_v[0, sl] += part

    start(0, 0, buf0, sem0)

    def tok_body(t, carry):
        wait(buf0, sem0)
        start(t, 1, buf1, sem1)
        combine(buf0, t, 0, True)
        wait(buf1, sem1)
        start(jnp.minimum(t + 1, TOK_PER_TILE - 1), 0, buf0, sem0)
        combine(buf1, t, 1, False)
        pltpu.sync_copy(acc_v, out_hbm.at[pl.ds(base + t, 1)])
        return carry

    lax.fori_loop(0, TOK_PER_TILE, tok_body, 0)
    wait(buf0, sem0)


def _sc_combine(values, midx, attn):
    mesh = plsc.VectorSubcoreMesh(core_axis_name="c", subcore_axis_name="s")
    f = functools.partial(
        pl.kernel,
        mesh=mesh,
        out_type=jax.ShapeDtypeStruct((SEQ, D_MODEL), jnp.float32),
        scratch_types=[
            pltpu.VMEM((TOK_PER_TILE, KH), jnp.int32),
            pltpu.VMEM((TOK_PER_TILE, KH), jnp.float32),
            pltpu.VMEM((HALF, D_MODEL), jnp.float32),
            pltpu.VMEM((HALF, D_MODEL), jnp.float32),
            pltpu.VMEM((1, D_MODEL), jnp.float32),
            pltpu.SemaphoreType.DMA,
            pltpu.SemaphoreType.DMA,
        ],
    )(_sc_body)
    return f(values, midx, attn)


def kernel(x, W_q, b_q, bn_gamma, bn_beta, c1, c2, values):
    bs, seq_len, d_model = x.shape
    x2d = x.reshape(seq_len, d_model)
    q, s1, s2 = _tc_proj(x2d, W_q, b_q.reshape(1, -1))
    midx_h, attn_h = _tc_scores(q, s1, s2, bn_gamma.reshape(1, -1),
                                bn_beta.reshape(1, -1), c1, c2)
    midx = jnp.transpose(midx_h, (1, 0, 2)).reshape(SEQ, KH)
    attn = jnp.transpose(attn_h, (1, 0, 2)).reshape(SEQ, KH)
    out = _sc_combine(values, midx, attn)
    return out.reshape(bs, seq_len, d_model)


# speedup vs baseline: 2.2239x; 1.1540x over previous
import functools

import jax
import jax.numpy as jnp
from jax import lax
from jax.experimental import pallas as pl
from jax.experimental.pallas import tpu as pltpu
from jax.experimental.pallas import tpu_sc as plsc

D_MODEL = 1024
QDIM = 512
NSUB = 256
K = 16
H = 4
SEQ = 2048
TOK_BLK = 256
N_BLK = SEQ // TOK_BLK
KH = K * H


def _proj_body(x_ref, w_ref, b_ref, q_ref, s1_ref, s2_ref):
    q = jnp.dot(x_ref[...], w_ref[...], preferred_element_type=jnp.float32)
    q = q + b_ref[...]
    q_ref[...] = q
    s1_ref[...] = jnp.sum(q, axis=0, keepdims=True)[None]
    s2_ref[...] = jnp.sum(q * q, axis=0, keepdims=True)[None]


def _tc_proj(x2d, W_q, b_q2d):
    return pl.pallas_call(
        _proj_body,
        grid=(N_BLK,),
        in_specs=[
            pl.BlockSpec((TOK_BLK, D_MODEL), lambda i: (i, 0)),
            pl.BlockSpec((D_MODEL, H * QDIM), lambda i: (0, 0)),
            pl.BlockSpec((1, H * QDIM), lambda i: (0, 0)),
        ],
        out_specs=[
            pl.BlockSpec((TOK_BLK, H * QDIM), lambda i: (i, 0)),
            pl.BlockSpec((1, 1, H * QDIM), lambda i: (i, 0, 0)),
            pl.BlockSpec((1, 1, H * QDIM), lambda i: (i, 0, 0)),
        ],
        out_shape=[
            jax.ShapeDtypeStruct((SEQ, H * QDIM), jnp.float32),
            jax.ShapeDtypeStruct((N_BLK, 1, H * QDIM), jnp.float32),
            jax.ShapeDtypeStruct((N_BLK, 1, H * QDIM), jnp.float32),
        ],
    )(x2d, W_q, b_q2d)


def _topk16(s, iota):
    T = s.shape[0]
    lane16 = lax.broadcasted_iota(jnp.int32, (T, K), 1)
    ts = jnp.zeros((T, K), jnp.float32)
    ti = jnp.zeros((T, K), jnp.int32)
    for k in range(K):
        m = jnp.max(s, axis=1, keepdims=True)
        hit = s == m
        arg = jnp.min(jnp.where(hit, iota, NSUB), axis=1, keepdims=True)
        ts = jnp.where(lane16 == k, m, ts)
        ti = jnp.where(lane16 == k, arg, ti)
        s = jnp.where(iota == arg, -jnp.inf, s)
    return ts, ti


def _scores_body(q_ref, s1_ref, s2_ref, g_ref, bb_ref, c1_ref, c2_ref,
                 midx_ref, attn_ref):
    inv_n = jnp.float32(1.0 / SEQ)
    mean = jnp.sum(s1_ref[...][:, 0, :], axis=0, keepdims=True) * inv_n
    ex2 = jnp.sum(s2_ref[...][:, 0, :], axis=0, keepdims=True) * inv_n
    var = ex2 - mean * mean
    scale = g_ref[...] / jnp.sqrt(var + 1e-5)
    shift = bb_ref[...] - mean * scale
    qn = q_ref[...] * scale + shift

    T = TOK_BLK
    iota = lax.broadcasted_iota(jnp.int32, (T, NSUB), 1)
    ri = lax.broadcasted_iota(jnp.int32, (K, NSUB), 0)
    ci = lax.broadcasted_iota(jnp.int32, (K, NSUB), 1)
    R1 = (ci // K == ri).astype(jnp.float32)
    R2 = (ci % K == ri).astype(jnp.float32)

    q1 = qn[:, :NSUB]
    q2 = qn[:, NSUB:]
    sc1 = lax.dot_general(q1, c1_ref[0], (((1,), (1,)), ((), ())),
                          preferred_element_type=jnp.float32)
    sc2 = lax.dot_general(q2, c2_ref[0], (((1,), (1,)), ((), ())),
                          preferred_element_type=jnp.float32)
    ts1, ti1 = _topk16(sc1, iota)
    ts2, ti2 = _topk16(sc2, iota)
    all_s = (jnp.dot(ts1, R1, preferred_element_type=jnp.float32,
                     precision=lax.Precision.HIGHEST)
             + jnp.dot(ts2, R2, preferred_element_type=jnp.float32,
                       precision=lax.Precision.HIGHEST))
    gs, gi = _topk16(all_s, iota)
    p1 = gi // K
    p2 = gi % K
    real1 = jnp.zeros((T, K), jnp.int32)
    real2 = jnp.zeros((T, K), jnp.int32)
    for p in range(K):
        real1 = real1 + jnp.where(p1 == p, ti1[:, p:p + 1], 0)
        real2 = real2 + jnp.where(p2 == p, ti2[:, p:p + 1], 0)
    mem = real1 * NSUB + real2
    m = jnp.max(gs, axis=1, keepdims=True)
    e = jnp.exp(gs - m)
    w = e / jnp.sum(e, axis=1, keepdims=True)
    midx_ref[0] = mem
    attn_ref[0] = w


def _tc_scores(q, s1, s2, gamma2d, beta2d, c1, c2):
    return pl.pallas_call(
        _scores_body,
        grid=(H, N_BLK),
        in_specs=[
            pl.BlockSpec((TOK_BLK, QDIM), lambda h, i: (i, h)),
            pl.BlockSpec((N_BLK, 1, QDIM), lambda h, i: (0, 0, h)),
            pl.BlockSpec((N_BLK, 1, QDIM), lambda h, i: (0, 0, h)),
            pl.BlockSpec((1, QDIM), lambda h, i: (0, h)),
            pl.BlockSpec((1, QDIM), lambda h, i: (0, h)),
            pl.BlockSpec((1, NSUB, QDIM // 2), lambda h, i: (h, 0, 0)),
            pl.BlockSpec((1, NSUB, QDIM // 2), lambda h, i: (h, 0, 0)),
        ],
        out_specs=[
            pl.BlockSpec((1, TOK_BLK, K), lambda h, i: (h, i, 0)),
            pl.BlockSpec((1, TOK_BLK, K), lambda h, i: (h, i, 0)),
        ],
        out_shape=[
            jax.ShapeDtypeStruct((H, SEQ, K), jnp.int32),
            jax.ShapeDtypeStruct((H, SEQ, K), jnp.float32),
        ],
    )(q, s1, s2, gamma2d, beta2d, c1, c2)


TOK_PER_TILE = SEQ // 32


QROWS = K
NBUF = 4


def _sc_body(values_hbm, idx_hbm, w_hbm, out_hbm, idx_v, w_v,
             buf0, buf1, buf2, buf3, acc_v, sem0, sem1, sem2, sem3):
    wid = lax.axis_index("s") * 2 + lax.axis_index("c")
    base = wid * TOK_PER_TILE
    pltpu.sync_copy(idx_hbm.at[pl.ds(base, TOK_PER_TILE)], idx_v)
    pltpu.sync_copy(w_hbm.at[pl.ds(base, TOK_PER_TILE)], w_v)

    bufs = [buf0, buf1, buf2, buf3]
    sems = [sem0, sem1, sem2, sem3]

    def start(t, q, buf, sem):
        pltpu.async_copy(
            values_hbm.at[idx_v.at[t, pl.ds(q * QROWS, QROWS)]], buf, sem)

    def wait(buf, sem):
        pltpu.make_async_copy(values_hbm.at[pl.ds(0, QROWS)], buf, sem).wait()

    def combine(buf, t, q, first):
        w_vec = w_v[t, pl.ds(q * QROWS, K)]
        ws = [w_vec[i] for i in range(K)]

        @plsc.parallel_loop(0, D_MODEL // 16, unroll=2)
        def j_body(j):
            sl = pl.ds(j * 16, 16)
            parts = [ws[p] * buf[p, sl] for p in range(4)]
            for i in range(4, K):
                p = i % 4
                parts[p] = parts[p] + ws[i] * buf[i, sl]
            part = (parts[0] + parts[1]) + (parts[2] + parts[3])
            if first:
                acc_v[0, sl] = part
            else:
                acc_v[0, sl] += part

    for k in range(NBUF):
        start(0, k, bufs[k], sems[k])

    def tok_body(t, carry):
        tn = jnp.minimum(t + 1, TOK_PER_TILE - 1)
        for k in range(NBUF):
            wait(bufs[k], sems[k])
            combine(bufs[k], t, k, first=(k == 0))
            start(tn, k, bufs[k], sems[k])
        pltpu.sync_copy(acc_v, out_hbm.at[pl.ds(base + t, 1)])
        return carry

    lax.fori_loop(0, TOK_PER_TILE, tok_body, 0)
    for k in range(NBUF):
        wait(bufs[k], sems[k])


def _sc_combine(values, midx, attn):
    mesh = plsc.VectorSubcoreMesh(core_axis_name="c", subcore_axis_name="s")
    f = functools.partial(
        pl.kernel,
        mesh=mesh,
        out_type=jax.ShapeDtypeStruct((SEQ, D_MODEL), jnp.float32),
        scratch_types=[
            pltpu.VMEM((TOK_PER_TILE, KH), jnp.int32),
            pltpu.VMEM((TOK_PER_TILE, KH), jnp.float32),
            pltpu.VMEM((QROWS, D_MODEL), jnp.float32),
            pltpu.VMEM((QROWS, D_MODEL), jnp.float32),
            pltpu.VMEM((QROWS, D_MODEL), jnp.float32),
            pltpu.VMEM((QROWS, D_MODEL), jnp.float32),
            pltpu.VMEM((1, D_MODEL), jnp.float32),
            pltpu.SemaphoreType.DMA,
            pltpu.SemaphoreType.DMA,
            pltpu.SemaphoreType.DMA,
            pltpu.SemaphoreType.DMA,
        ],
    )(_sc_body)
    return f(values, midx, attn)


def kernel(x, W_q, b_q, bn_gamma, bn_beta, c1, c2, values):
    bs, seq_len, d_model = x.shape
    x2d = x.reshape(seq_len, d_model)
    q, s1, s2 = _tc_proj(x2d, W_q, b_q.reshape(1, -1))
    midx_h, attn_h = _tc_scores(q, s1, s2, bn_gamma.reshape(1, -1),
                                bn_beta.reshape(1, -1), c1, c2)
    midx = jnp.transpose(midx_h, (1, 0, 2)).reshape(SEQ, KH)
    attn = jnp.transpose(attn_h, (1, 0, 2)).reshape(SEQ, KH)
    out = _sc_combine(values, midx, attn)
    return out.reshape(bs, seq_len, d_model)
